# baseline (speedup 1.0000x reference)
import functools

import jax
import jax.numpy as jnp
from jax import lax
from jax.experimental import pallas as pl
from jax.experimental.pallas import tpu as pltpu
from jax.experimental.pallas import tpu_sc as plsc

_N = 100000
_E = 6400000
_F = 16
_EC = 128
_U = _E // (8 * _EC)
_NW = 32
_WR = 6248
_WREM = _N - 16 * _WR


_SB = 13


def _fire_gathers(h_hbm, idx, rows, gsem, uu, p):
    for b in range(8):
        pltpu.async_copy(h_hbm.at[idx.at[uu, b]], rows.at[p, b], gsem)


def _fire_scatters(acc, idx, rows, ssem, uu, p):
    for b in range(8):
        pltpu.async_copy(rows.at[p, b], acc.at[idx.at[uu, b]], ssem, add=True)


def _drain(rows, sem, hbm_dummy, n):
    for _ in range(n):
        for b in range(8):
            pltpu.make_async_copy(hbm_dummy.at[pl.ds(0, _EC)],
                                  rows.at[0, b], sem).wait()


def _edge_range(h_hbm, ei_hbm, sidx, didx, rows, acc, gsem, ssem,
                q0, nsb):
    @pl.loop(0, nsb)
    def _sb(t):
        q = q0 + t * _SB
        pltpu.sync_copy(ei_hbm.at[0, pl.ds(q, _SB)], sidx)
        pltpu.sync_copy(ei_hbm.at[1, pl.ds(q, _SB)], didx)
        _fire_gathers(h_hbm, sidx, rows, gsem, 0, 0)

        @pl.loop(0, _SB)
        def _u(u):
            p = lax.rem(u, 2)

            @pl.when(u > 0)
            def _ws():
                _drain(rows, ssem, h_hbm, 1)

            @pl.when(u < _SB - 1)
            def _fg():
                _fire_gathers(h_hbm, sidx, rows, gsem, u + 1, 1 - p)

            _drain(rows, gsem, h_hbm, 1)
            _fire_scatters(acc, didx, rows, ssem, u, p)

        _drain(rows, ssem, h_hbm, 1)


def _one_unit(h_hbm, ei_hbm, sidx, didx, rows, acc, gsem, ssem, q):
    pltpu.sync_copy(ei_hbm.at[0, pl.ds(q, 1)], sidx.at[pl.ds(0, 1)])
    pltpu.sync_copy(ei_hbm.at[1, pl.ds(q, 1)], didx.at[pl.ds(0, 1)])
    _fire_gathers(h_hbm, sidx, rows, gsem, 0, 0)
    _drain(rows, gsem, h_hbm, 1)
    _fire_scatters(acc, didx, rows, ssem, 0, 0)
    _drain(rows, ssem, h_hbm, 1)


def _zero_acc(zini_hbm, acc, s):
    pltpu.sync_copy(zini_hbm, acc.at[pl.ds(s * _WR, _WR)])

    @pl.when(s == 15)
    def _zrem():
        pltpu.sync_copy(zini_hbm.at[pl.ds(0, _WREM)],
                        acc.at[pl.ds(16 * _WR, _WREM)])


def _write_out(acc, out_hbm, c, s):
    pltpu.sync_copy(acc.at[pl.ds(s * _WR, _WR)],
                    out_hbm.at[c, pl.ds(s * _WR, _WR)])

    @pl.when(s == 15)
    def _wrem():
        pltpu.sync_copy(acc.at[pl.ds(16 * _WR, _WREM)],
                        out_hbm.at[c, pl.ds(16 * _WR, _WREM)])


_UPS = _U // 16
_LU2 = _U - 16 * _UPS


def _sc_scatter_body(ha_hbm, hb_hbm, ei_hbm, zini_hbm, out_hbm,
                     sidx, didx, rows, acc, gsem, ssem):
    c = lax.axis_index("c")
    s = lax.axis_index("s")
    _zero_acc(zini_hbm, acc, s)
    plsc.subcore_barrier()

    q0 = s * _UPS

    @pl.when(c == 0)
    def _feat_lo():
        _edge_range(ha_hbm, ei_hbm, sidx, didx, rows, acc,
                    gsem, ssem, q0, _UPS // _SB)

        @pl.when(s < _LU2)
        def _extra():
            _one_unit(ha_hbm, ei_hbm, sidx, didx, rows, acc,
                      gsem, ssem, 16 * _UPS + s)

    @pl.when(c == 1)
    def _feat_hi():
        _edge_range(hb_hbm, ei_hbm, sidx, didx, rows, acc,
                    gsem, ssem, q0, _UPS // _SB)

        @pl.when(s < _LU2)
        def _extra():
            _one_unit(hb_hbm, ei_hbm, sidx, didx, rows, acc,
                      gsem, ssem, 16 * _UPS + s)

    plsc.subcore_barrier()
    _write_out(acc, out_hbm, c, s)


_sc_scatter = functools.partial(
    pl.kernel,
    out_type=jax.ShapeDtypeStruct((2, _N, 8), jnp.float32),
    mesh=plsc.VectorSubcoreMesh(core_axis_name="c", subcore_axis_name="s"),
    compiler_params=pltpu.CompilerParams(use_tc_tiling_on_sc=False),
    scratch_types=[
        pltpu.VMEM((_SB, 8, _EC), jnp.int32),
        pltpu.VMEM((_SB, 8, _EC), jnp.int32),
        pltpu.VMEM((2, 8, _EC, 8), jnp.float32),
        pltpu.VMEM_SHARED((_N, 8), jnp.float32),
        pltpu.SemaphoreType.DMA,
        pltpu.SemaphoreType.DMA,
    ],
)(_sc_scatter_body)


_B = 1000
_P = _N // 4


def _prep_body(x_ref, slo_ref, shi_ref, xa_ref, xb_ref):
    xb = x_ref[...]
    xa_ref[...] = jnp.dot(xb, slo_ref[...], preferred_element_type=jnp.float32,
                 precision=lax.Precision.HIGHEST)
    xb_ref[...] = jnp.dot(xb, shi_ref[...], preferred_element_type=jnp.float32,
                 precision=lax.Precision.HIGHEST)


def _prep(xp, slo, shi):
    return pl.pallas_call(
        _prep_body,
        grid=(_P // _B,),
        in_specs=[
            pl.BlockSpec((_B, 44), lambda i: (i, 0)),
            pl.BlockSpec((44, 32), lambda i: (0, 0)),
            pl.BlockSpec((44, 32), lambda i: (0, 0)),
        ],
        out_specs=[
            pl.BlockSpec((_B, 32), lambda i: (i, 0)),
            pl.BlockSpec((_B, 32), lambda i: (i, 0)),
        ],
        out_shape=[
            jax.ShapeDtypeStruct((_P, 32), jnp.float32),
            jax.ShapeDtypeStruct((_P, 32), jnp.float32),
        ],
    )(xp, slo, shi)


def _dense1_body(x_ref, a0_ref, a1_ref, wi_ref, bi_ref, wnlo_ref, wnhi_ref,
                 bn_ref, slo_ref, shi_ref, h1_ref, ha_ref, hb_ref):
    hi = jnp.dot(x_ref[...], wi_ref[...],
                 preferred_element_type=jnp.float32,
                 precision=lax.Precision.HIGHEST) + bi_ref[...]
    hn = (jnp.dot(a0_ref[...], wnlo_ref[...],
                  preferred_element_type=jnp.float32,
                 precision=lax.Precision.HIGHEST)
          + jnp.dot(a1_ref[...], wnhi_ref[...],
                    preferred_element_type=jnp.float32,
                 precision=lax.Precision.HIGHEST)) + bn_ref[...]
    h1 = jnp.maximum(hi, 0.0) + jnp.maximum(hn, 0.0)
    h1_ref[...] = h1
    ha_ref[...] = jnp.dot(h1, slo_ref[...], preferred_element_type=jnp.float32,
                 precision=lax.Precision.HIGHEST)
    hb_ref[...] = jnp.dot(h1, shi_ref[...], preferred_element_type=jnp.float32,
                 precision=lax.Precision.HIGHEST)


def _dense1(xp, a0, a1, wi, bi, wnlo, wnhi, bn, slo, shi):
    return pl.pallas_call(
        _dense1_body,
        grid=(_P // _B,),
        in_specs=[
            pl.BlockSpec((_B, 44), lambda i: (i, 0)),
            pl.BlockSpec((_B, 32), lambda i: (i, 0)),
            pl.BlockSpec((_B, 32), lambda i: (i, 0)),
            pl.BlockSpec((44, 64), lambda i: (0, 0)),
            pl.BlockSpec((1, 64), lambda i: (0, 0)),
            pl.BlockSpec((32, 64), lambda i: (0, 0)),
            pl.BlockSpec((32, 64), lambda i: (0, 0)),
            pl.BlockSpec((1, 64), lambda i: (0, 0)),
            pl.BlockSpec((64, 32), lambda i: (0, 0)),
            pl.BlockSpec((64, 32), lambda i: (0, 0)),
        ],
        out_specs=[
            pl.BlockSpec((_B, 64), lambda i: (i, 0)),
            pl.BlockSpec((_B, 32), lambda i: (i, 0)),
            pl.BlockSpec((_B, 32), lambda i: (i, 0)),
        ],
        out_shape=[
            jax.ShapeDtypeStruct((_P, 64), jnp.float32),
            jax.ShapeDtypeStruct((_P, 32), jnp.float32),
            jax.ShapeDtypeStruct((_P, 32), jnp.float32),
        ],
    )(xp, a0, a1, wi, bi, wnlo, wnhi, bn, slo, shi)


def _dense2_body(h1_ref, a0_ref, a1_ref, wi_ref, bi_ref, wnlo_ref, wnhi_ref,
                 bn_ref, walo_ref, wahi_ref, wdlo_ref, wdhi_ref, batt_ref,
                 e_ref, u_ref, ssum_ref):
    i = pl.program_id(0)
    zi = jnp.maximum(
        jnp.dot(h1_ref[...], wi_ref[...],
                preferred_element_type=jnp.float32,
                 precision=lax.Precision.HIGHEST) + bi_ref[...], 0.0)
    zn = jnp.maximum(
        jnp.dot(a0_ref[...], wnlo_ref[...],
                preferred_element_type=jnp.float32,
                 precision=lax.Precision.HIGHEST)
        + jnp.dot(a1_ref[...], wnhi_ref[...],
                  preferred_element_type=jnp.float32,
                 precision=lax.Precision.HIGHEST) + bn_ref[...], 0.0)
    t = (jnp.dot(zi, walo_ref[...], preferred_element_type=jnp.float32,
                 precision=lax.Precision.HIGHEST)
         + jnp.dot(zn, wahi_ref[...], preferred_element_type=jnp.float32,
                 precision=lax.Precision.HIGHEST))
    e = jnp.exp(jnp.tanh(t + batt_ref[0, 0]))
    u = (jnp.dot(zi, wdlo_ref[...], preferred_element_type=jnp.float32,
                 precision=lax.Precision.HIGHEST)
         + jnp.dot(zn, wdhi_ref[...], preferred_element_type=jnp.float32,
                 precision=lax.Precision.HIGHEST))
    e_ref[...] = e
    u_ref[...] = u

    @pl.when(i == 0)
    def _init():
        ssum_ref[0, 0] = 0.0

    ssum_ref[0, 0] += jnp.sum(e)


def _dense2(h1, a0, a1, wi, bi, wnlo, wnhi, bn, walo, wahi, wdlo, wdhi, batt):
    return pl.pallas_call(
        _dense2_body,
        grid=(_P // _B,),
        in_specs=[
            pl.BlockSpec((_B, 64), lambda i: (i, 0)),
            pl.BlockSpec((_B, 32), lambda i: (i, 0)),
            pl.BlockSpec((_B, 32), lambda i: (i, 0)),
            pl.BlockSpec((64, 128), lambda i: (0, 0)),
            pl.BlockSpec((1, 128), lambda i: (0, 0)),
            pl.BlockSpec((32, 128), lambda i: (0, 0)),
            pl.BlockSpec((32, 128), lambda i: (0, 0)),
            pl.BlockSpec((1, 128), lambda i: (0, 0)),
            pl.BlockSpec((128, 4), lambda i: (0, 0)),
            pl.BlockSpec((128, 4), lambda i: (0, 0)),
            pl.BlockSpec((128, 4), lambda i: (0, 0)),
            pl.BlockSpec((128, 4), lambda i: (0, 0)),
            pl.BlockSpec(memory_space=pltpu.SMEM),
        ],
        out_specs=[
            pl.BlockSpec((_B, 4), lambda i: (i, 0)),
            pl.BlockSpec((_B, 4), lambda i: (i, 0)),
            pl.BlockSpec(memory_space=pltpu.SMEM),
        ],
        out_shape=[
            jax.ShapeDtypeStruct((_P, 4), jnp.float32),
            jax.ShapeDtypeStruct((_P, 4), jnp.float32),
            jax.ShapeDtypeStruct((1, 1), jnp.float32),
        ],
    )(h1, a0, a1, wi, bi, wnlo, wnhi, bn, walo, wahi, wdlo, wdhi, batt)


def _final_body(e_ref, u_ref, ssum_ref, bd_ref, o_ref):
    o_ref[...] = (e_ref[...] * u_ref[...] * (1.0 / ssum_ref[0, 0])
                  + bd_ref[0, 0])


def _final(e, u, ssum, bd):
    return pl.pallas_call(
        _final_body,
        grid=(_P // _B,),
        in_specs=[
            pl.BlockSpec((_B, 4), lambda i: (i, 0)),
            pl.BlockSpec((_B, 4), lambda i: (i, 0)),
            pl.BlockSpec(memory_space=pltpu.SMEM),
            pl.BlockSpec(memory_space=pltpu.SMEM),
        ],
        out_specs=pl.BlockSpec((_B, 4), lambda i: (i, 0)),
        out_shape=jax.ShapeDtypeStruct((_P, 4), jnp.float32),
    )(e, u, ssum, bd)


def _kron4(w):
    return jnp.kron(jnp.eye(4, dtype=jnp.float32), w.astype(jnp.float32))


def kernel(x, edge_index, W1_int, b1_int, W1_nh, b1_nh,
           W2_int, b2_int, W2_nh, b2_nh, w_att, b_att, W_d, b_d):
    xp = x.reshape(_P, 44)
    ei = edge_index.reshape(2, _U, 8, _EC)
    zini = jnp.zeros((_WR, 8), jnp.float32)

    slo11 = _kron4(jnp.eye(11, 8, dtype=jnp.float32))
    shi11 = _kron4(jnp.eye(11, 8, k=-8, dtype=jnp.float32))
    slo16 = _kron4(jnp.eye(16, 8, dtype=jnp.float32))
    shi16 = _kron4(jnp.eye(16, 8, k=-8, dtype=jnp.float32))
    w1i = _kron4(W1_int)
    w1nlo = _kron4(W1_nh[:8])
    w1nhi = _kron4(jnp.pad(W1_nh[8:], ((0, 5), (0, 0))))
    w2i = _kron4(W2_int)
    w2nlo = _kron4(W2_nh[:8])
    w2nhi = _kron4(W2_nh[8:])
    walo = _kron4(w_att[:32].reshape(32, 1))
    wahi = _kron4(w_att[32:].reshape(32, 1))
    wdlo = _kron4(W_d[:32])
    wdhi = _kron4(W_d[32:])
    b1t = jnp.tile(b1_int, 4).reshape(1, 64)
    b1nt = jnp.tile(b1_nh, 4).reshape(1, 64)
    b2t = jnp.tile(b2_int, 4).reshape(1, 128)
    b2nt = jnp.tile(b2_nh, 4).reshape(1, 128)

    xa, xb = _prep(xp, slo11, shi11)
    agg1 = _sc_scatter(xa.reshape(_N, 8), xb.reshape(_N, 8), ei, zini)
    h1, h1a, h1b = _dense1(xp, agg1[0].reshape(_P, 32), agg1[1].reshape(_P, 32),
                           w1i, b1t, w1nlo, w1nhi, b1nt, slo16, shi16)
    agg2 = _sc_scatter(h1a.reshape(_N, 8), h1b.reshape(_N, 8), ei, zini)
    e, u, ssum = _dense2(h1, agg2[0].reshape(_P, 32), agg2[1].reshape(_P, 32),
                         w2i, b2t, w2nlo, w2nhi, b2nt,
                         walo, wahi, wdlo, wdhi, b_att.reshape(1, 1))
    out = _final(e, u, ssum, b_d.reshape(1, 1))
    return out.reshape(_N)


# speedup vs baseline: 1.3229x; 1.3229x over previous
import functools

import jax
import jax.numpy as jnp
from jax import lax
from jax.experimental import pallas as pl
from jax.experimental.pallas import tpu as pltpu
from jax.experimental.pallas import tpu_sc as plsc

_N = 100000
_E = 6400000
_F = 16
_EC = 128
_U = _E // (8 * _EC)
_NW = 32
_WR = 6248
_WREM = _N - 16 * _WR


_SB = 13


def _fire_gathers(h_hbm, idx, rows, gsem, uu, p):
    for b in range(8):
        pltpu.async_copy(h_hbm.at[idx.at[uu, b]], rows.at[p, b], gsem)


def _fire_scatters(acc, idx, rows, ssem, uu, p):
    for b in range(8):
        pltpu.async_copy(rows.at[p, b], acc.at[idx.at[uu, b]], ssem, add=True)


def _drain(rows, sem, hbm_dummy, n):
    for _ in range(n):
        for b in range(8):
            pltpu.make_async_copy(hbm_dummy.at[pl.ds(0, _EC)],
                                  rows.at[0, b], sem).wait()


def _edge_range(h_hbm, ei_hbm, sidx, didx, rows, acc, gsem, ssem,
                q0, nsb):
    @pl.loop(0, nsb)
    def _sb(t):
        q = q0 + t * _SB
        pltpu.sync_copy(ei_hbm.at[0, pl.ds(q, _SB)], sidx)
        pltpu.sync_copy(ei_hbm.at[1, pl.ds(q, _SB)], didx)
        _fire_gathers(h_hbm, sidx, rows, gsem, 0, 0)

        @pl.loop(0, _SB)
        def _u(u):
            p = lax.rem(u, 2)

            @pl.when(u > 0)
            def _ws():
                _drain(rows, ssem, h_hbm, 1)

            @pl.when(u < _SB - 1)
            def _fg():
                _fire_gathers(h_hbm, sidx, rows, gsem, u + 1, 1 - p)

            _drain(rows, gsem, h_hbm, 1)
            _fire_scatters(acc, didx, rows, ssem, u, p)

        _drain(rows, ssem, h_hbm, 1)


def _one_unit(h_hbm, ei_hbm, sidx, didx, rows, acc, gsem, ssem, q):
    pltpu.sync_copy(ei_hbm.at[0, pl.ds(q, 1)], sidx.at[pl.ds(0, 1)])
    pltpu.sync_copy(ei_hbm.at[1, pl.ds(q, 1)], didx.at[pl.ds(0, 1)])
    _fire_gathers(h_hbm, sidx, rows, gsem, 0, 0)
    _drain(rows, gsem, h_hbm, 1)
    _fire_scatters(acc, didx, rows, ssem, 0, 0)
    _drain(rows, ssem, h_hbm, 1)


def _zero_acc(zini_hbm, acc, s):
    pltpu.sync_copy(zini_hbm, acc.at[pl.ds(s * _WR, _WR)])

    @pl.when(s == 15)
    def _zrem():
        pltpu.sync_copy(zini_hbm.at[pl.ds(0, _WREM)],
                        acc.at[pl.ds(16 * _WR, _WREM)])


def _write_out(acc, out_hbm, c, s):
    pltpu.sync_copy(acc.at[pl.ds(s * _WR, _WR)],
                    out_hbm.at[c, pl.ds(s * _WR, _WR)])

    @pl.when(s == 15)
    def _wrem():
        pltpu.sync_copy(acc.at[pl.ds(16 * _WR, _WREM)],
                        out_hbm.at[c, pl.ds(16 * _WR, _WREM)])


_UPS = _U // 16
_LU2 = _U - 16 * _UPS


def _sc_scatter_body(ha_hbm, hb_hbm, ei_hbm, zini_hbm, out_hbm,
                     sidx, didx, rows, acc, gsem, ssem):
    c = lax.axis_index("c")
    s = lax.axis_index("s")
    _zero_acc(zini_hbm, acc, s)
    plsc.subcore_barrier()

    q0 = s * _UPS

    @pl.when(c == 0)
    def _feat_lo():
        _edge_range(ha_hbm, ei_hbm, sidx, didx, rows, acc,
                    gsem, ssem, q0, _UPS // _SB)

        @pl.when(s < _LU2)
        def _extra():
            _one_unit(ha_hbm, ei_hbm, sidx, didx, rows, acc,
                      gsem, ssem, 16 * _UPS + s)

    @pl.when(c == 1)
    def _feat_hi():
        _edge_range(hb_hbm, ei_hbm, sidx, didx, rows, acc,
                    gsem, ssem, q0, _UPS // _SB)

        @pl.when(s < _LU2)
        def _extra():
            _one_unit(hb_hbm, ei_hbm, sidx, didx, rows, acc,
                      gsem, ssem, 16 * _UPS + s)

    plsc.subcore_barrier()
    _write_out(acc, out_hbm, c, s)


_sc_scatter = functools.partial(
    pl.kernel,
    out_type=jax.ShapeDtypeStruct((2, _N, 8), jnp.float32),
    mesh=plsc.VectorSubcoreMesh(core_axis_name="c", subcore_axis_name="s"),
    compiler_params=pltpu.CompilerParams(use_tc_tiling_on_sc=False),
    scratch_types=[
        pltpu.VMEM((_SB, 8, _EC), jnp.int32),
        pltpu.VMEM((_SB, 8, _EC), jnp.int32),
        pltpu.VMEM((2, 8, _EC, 8), jnp.float32),
        pltpu.VMEM_SHARED((_N, 8), jnp.float32),
        pltpu.SemaphoreType.DMA,
        pltpu.SemaphoreType.DMA,
    ],
)(_sc_scatter_body)


_B = 1000
_P = _N // 4


def _prep_body(x_ref, slo_ref, shi_ref, xa_ref, xb_ref):
    xb = x_ref[...]
    xa_ref[...] = jnp.dot(xb, slo_ref[...], preferred_element_type=jnp.float32,
                          precision=lax.Precision.HIGHEST)
    xb_ref[...] = jnp.dot(xb, shi_ref[...], preferred_element_type=jnp.float32,
                          precision=lax.Precision.HIGHEST)


def _prep(xp, slo, shi):
    return pl.pallas_call(
        _prep_body,
        grid=(_P // _B,),
        in_specs=[
            pl.BlockSpec((_B, 44), lambda i: (i, 0)),
            pl.BlockSpec((44, 32), lambda i: (0, 0)),
            pl.BlockSpec((44, 32), lambda i: (0, 0)),
        ],
        out_specs=[
            pl.BlockSpec((_B, 32), lambda i: (i, 0)),
            pl.BlockSpec((_B, 32), lambda i: (i, 0)),
        ],
        out_shape=[
            jax.ShapeDtypeStruct((_P, 32), jnp.float32),
            jax.ShapeDtypeStruct((_P, 32), jnp.float32),
        ],
    )(xp, slo, shi)


def _dense1_body(x_ref, a_ref, wi_ref, bi_ref, wnlo_ref, wnhi_ref, bn_ref,
                 slo_ref, shi_ref, h1_ref, ha_ref, hb_ref):
    hi = jnp.dot(x_ref[...], wi_ref[...],
                 preferred_element_type=jnp.float32) + bi_ref[...]
    hn = (jnp.dot(a_ref[0], wnlo_ref[...],
                  preferred_element_type=jnp.float32)
          + jnp.dot(a_ref[1], wnhi_ref[...],
                    preferred_element_type=jnp.float32)) + bn_ref[...]
    h1 = jnp.maximum(hi, 0.0) + jnp.maximum(hn, 0.0)
    h1_ref[...] = h1
    ha_ref[...] = jnp.dot(h1, slo_ref[...], preferred_element_type=jnp.float32,
                          precision=lax.Precision.HIGHEST)
    hb_ref[...] = jnp.dot(h1, shi_ref[...], preferred_element_type=jnp.float32,
                          precision=lax.Precision.HIGHEST)


def _dense1(xp, agg, wi, bi, wnlo, wnhi, bn, slo, shi):
    return pl.pallas_call(
        _dense1_body,
        grid=(_P // _B,),
        in_specs=[
            pl.BlockSpec((_B, 44), lambda i: (i, 0)),
            pl.BlockSpec((2, _B, 32), lambda i: (0, i, 0)),
            pl.BlockSpec((44, 64), lambda i: (0, 0)),
            pl.BlockSpec((1, 64), lambda i: (0, 0)),
            pl.BlockSpec((32, 64), lambda i: (0, 0)),
            pl.BlockSpec((32, 64), lambda i: (0, 0)),
            pl.BlockSpec((1, 64), lambda i: (0, 0)),
            pl.BlockSpec((64, 32), lambda i: (0, 0)),
            pl.BlockSpec((64, 32), lambda i: (0, 0)),
        ],
        out_specs=[
            pl.BlockSpec((_B, 64), lambda i: (i, 0)),
            pl.BlockSpec((_B, 32), lambda i: (i, 0)),
            pl.BlockSpec((_B, 32), lambda i: (i, 0)),
        ],
        out_shape=[
            jax.ShapeDtypeStruct((_P, 64), jnp.float32),
            jax.ShapeDtypeStruct((_P, 32), jnp.float32),
            jax.ShapeDtypeStruct((_P, 32), jnp.float32),
        ],
    )(xp, agg, wi, bi, wnlo, wnhi, bn, slo, shi)


def _dense2_body(h1_ref, a_ref, wi_ref, bi_ref, wnlo_ref, wnhi_ref, bn_ref,
                 walo_ref, wahi_ref, wdlo_ref, wdhi_ref, batt_ref,
                 e_ref, u_ref, ssum_ref):
    i = pl.program_id(0)
    zi = jnp.maximum(
        jnp.dot(h1_ref[...], wi_ref[...],
                preferred_element_type=jnp.float32) + bi_ref[...], 0.0)
    zn = jnp.maximum(
        jnp.dot(a_ref[0], wnlo_ref[...],
                preferred_element_type=jnp.float32)
        + jnp.dot(a_ref[1], wnhi_ref[...],
                  preferred_element_type=jnp.float32) + bn_ref[...], 0.0)
    t = (jnp.dot(zi, walo_ref[...], preferred_element_type=jnp.float32)
         + jnp.dot(zn, wahi_ref[...], preferred_element_type=jnp.float32))
    e = jnp.exp(jnp.tanh(t + batt_ref[0, 0]))
    u = (jnp.dot(zi, wdlo_ref[...], preferred_element_type=jnp.float32)
         + jnp.dot(zn, wdhi_ref[...], preferred_element_type=jnp.float32))
    e_ref[...] = e
    u_ref[...] = u

    @pl.when(i == 0)
    def _init():
        ssum_ref[0, 0] = 0.0

    ssum_ref[0, 0] += jnp.sum(e)


def _dense2(h1, agg, wi, bi, wnlo, wnhi, bn, walo, wahi, wdlo, wdhi, batt):
    return pl.pallas_call(
        _dense2_body,
        grid=(_P // _B,),
        in_specs=[
            pl.BlockSpec((_B, 64), lambda i: (i, 0)),
            pl.BlockSpec((2, _B, 32), lambda i: (0, i, 0)),
            pl.BlockSpec((64, 128), lambda i: (0, 0)),
            pl.BlockSpec((1, 128), lambda i: (0, 0)),
            pl.BlockSpec((32, 128), lambda i: (0, 0)),
            pl.BlockSpec((32, 128), lambda i: (0, 0)),
            pl.BlockSpec((1, 128), lambda i: (0, 0)),
            pl.BlockSpec((128, 4), lambda i: (0, 0)),
            pl.BlockSpec((128, 4), lambda i: (0, 0)),
            pl.BlockSpec((128, 4), lambda i: (0, 0)),
            pl.BlockSpec((128, 4), lambda i: (0, 0)),
            pl.BlockSpec(memory_space=pltpu.SMEM),
        ],
        out_specs=[
            pl.BlockSpec((_B, 4), lambda i: (i, 0)),
            pl.BlockSpec((_B, 4), lambda i: (i, 0)),
            pl.BlockSpec(memory_space=pltpu.SMEM),
        ],
        out_shape=[
            jax.ShapeDtypeStruct((_P, 4), jnp.float32),
            jax.ShapeDtypeStruct((_P, 4), jnp.float32),
            jax.ShapeDtypeStruct((1, 1), jnp.float32),
        ],
    )(h1, agg, wi, bi, wnlo, wnhi, bn, walo, wahi, wdlo, wdhi, batt)


def _final_body(e_ref, u_ref, ssum_ref, bd_ref, o_ref):
    o_ref[...] = (e_ref[...] * u_ref[...] * (1.0 / ssum_ref[0, 0])
                  + bd_ref[0, 0])


def _final(e, u, ssum, bd):
    return pl.pallas_call(
        _final_body,
        grid=(_P // _B,),
        in_specs=[
            pl.BlockSpec((_B, 4), lambda i: (i, 0)),
            pl.BlockSpec((_B, 4), lambda i: (i, 0)),
            pl.BlockSpec(memory_space=pltpu.SMEM),
            pl.BlockSpec(memory_space=pltpu.SMEM),
        ],
        out_specs=pl.BlockSpec((_B, 4), lambda i: (i, 0)),
        out_shape=jax.ShapeDtypeStruct((_P, 4), jnp.float32),
    )(e, u, ssum, bd)


def _kron4(w):
    return jnp.kron(jnp.eye(4, dtype=jnp.float32), w.astype(jnp.float32))


def kernel(x, edge_index, W1_int, b1_int, W1_nh, b1_nh,
           W2_int, b2_int, W2_nh, b2_nh, w_att, b_att, W_d, b_d):
    xp = x.reshape(_P, 44)
    ei = edge_index.reshape(2, _U, 8, _EC)
    zini = jnp.zeros((_WR, 8), jnp.float32)

    slo11 = _kron4(jnp.eye(11, 8, dtype=jnp.float32))
    shi11 = _kron4(jnp.eye(11, 8, k=-8, dtype=jnp.float32))
    slo16 = _kron4(jnp.eye(16, 8, dtype=jnp.float32))
    shi16 = _kron4(jnp.eye(16, 8, k=-8, dtype=jnp.float32))
    w1i = _kron4(W1_int)
    w1nlo = _kron4(W1_nh[:8])
    w1nhi = _kron4(jnp.pad(W1_nh[8:], ((0, 5), (0, 0))))
    w2i = _kron4(W2_int)
    w2nlo = _kron4(W2_nh[:8])
    w2nhi = _kron4(W2_nh[8:])
    walo = _kron4(w_att[:32].reshape(32, 1))
    wahi = _kron4(w_att[32:].reshape(32, 1))
    wdlo = _kron4(W_d[:32])
    wdhi = _kron4(W_d[32:])
    b1t = jnp.tile(b1_int, 4).reshape(1, 64)
    b1nt = jnp.tile(b1_nh, 4).reshape(1, 64)
    b2t = jnp.tile(b2_int, 4).reshape(1, 128)
    b2nt = jnp.tile(b2_nh, 4).reshape(1, 128)

    xa, xb = _prep(xp, slo11, shi11)
    agg1 = _sc_scatter(xa.reshape(_N, 8), xb.reshape(_N, 8), ei, zini)
    h1, h1a, h1b = _dense1(xp, agg1.reshape(2, _P, 32), w1i, b1t, w1nlo, w1nhi, b1nt,
                           slo16, shi16)
    agg2 = _sc_scatter(h1a.reshape(_N, 8), h1b.reshape(_N, 8), ei, zini)
    e, u, ssum = _dense2(h1, agg2.reshape(2, _P, 32), w2i, b2t, w2nlo, w2nhi, b2nt,
                         walo, wahi, wdlo, wdhi, b_att.reshape(1, 1))
    out = _final(e, u, ssum, b_d.reshape(1, 1))
    return out.reshape(_N)
